# Initial kernel scaffold; baseline (speedup 1.0000x reference)
import jax
import jax.numpy as jnp
from jax.experimental import pallas as pl


def kernel(h, ffn_nw, w13, w2, attn_nw):
    raise NotImplementedError("write your pallas kernel here")


# speedup vs baseline: 1.2967x; 1.2967x over previous
import functools

import jax
import jax.numpy as jnp
from jax.experimental import pallas as pl
from jax.experimental.pallas import tpu as pltpu


def _round_up(x, m):
    return (x + m - 1) // m * m


def _ffn_block_kernel(h_ref, fnw_ref, w13_ref, w2_ref, anw_ref,
                      o_ref, x_ref, *, eps, inv_dim):
    k = pl.program_id(1)
    th = w2_ref.shape[0]

    @pl.when(k == 0)
    def _init():
        h = h_ref[...]
        ms = jnp.sum(h * h, axis=-1, keepdims=True) * inv_dim
        x_ref[...] = (h * jax.lax.rsqrt(ms + eps) * fnw_ref[...]).astype(x_ref.dtype)
        o_ref[...] = h

    hh = jnp.dot(x_ref[...], w13_ref[...], preferred_element_type=jnp.float32)
    gated = jax.nn.silu(hh[:, :th]) * hh[:, th:]
    o_ref[...] += jnp.dot(gated.astype(w2_ref.dtype), w2_ref[...],
                          preferred_element_type=jnp.float32)

    @pl.when(k == pl.num_programs(1) - 1)
    def _finalize():
        y = o_ref[...]
        ms2 = jnp.sum(y * y, axis=-1, keepdims=True) * inv_dim
        o_ref[...] = y * jax.lax.rsqrt(ms2 + eps) * anw_ref[...]


def kernel(h, ffn_nw, w13, w2, attn_nw, *, eps=1e-6):
    B, S, dim = h.shape
    dim_p = ffn_nw.shape[1]
    th = 256
    nk = w13.shape[1] // (2 * th)
    tokens = B * S

    tm = 512
    while tokens % tm and tm > 8:
        tm //= 2
    tokens_p = _round_up(tokens, tm)
    n_tiles = tokens_p // tm

    h2d = h.reshape(tokens, dim)
    if tokens_p != tokens or dim_p != dim:
        h2d = jnp.pad(h2d, ((0, tokens_p - tokens), (0, dim_p - dim)))

    w_bytes = (w13.size + w2.size) * w13.dtype.itemsize
    cost = pl.CostEstimate(
        flops=int(6 * tokens_p * dim_p * (nk * th)),
        transcendentals=int(tokens_p * nk * th + 2 * tokens_p),
        bytes_accessed=int(w_bytes * n_tiles + 2 * tokens_p * dim_p * 4),
    )

    body = functools.partial(_ffn_block_kernel, eps=eps, inv_dim=1.0 / dim)

    out = pl.pallas_call(
        body,
        out_shape=jax.ShapeDtypeStruct((tokens_p, dim_p), h.dtype),
        grid=(n_tiles, nk),
        in_specs=[
            pl.BlockSpec((tm, dim_p), lambda i, k: (i, 0)),
            pl.BlockSpec((1, dim_p), lambda i, k: (0, 0)),
            pl.BlockSpec((dim_p, 2 * th), lambda i, k: (0, k)),
            pl.BlockSpec((th, dim_p), lambda i, k: (k, 0)),
            pl.BlockSpec((1, dim_p), lambda i, k: (0, 0)),
        ],
        out_specs=pl.BlockSpec((tm, dim_p), lambda i, k: (i, 0)),
        scratch_shapes=[pltpu.VMEM((tm, dim_p), w13.dtype)],
        compiler_params=pltpu.CompilerParams(
            dimension_semantics=("parallel", "arbitrary"),
            vmem_limit_bytes=60 * 1024 * 1024,
        ),
        cost_estimate=cost,
    )(h2d, ffn_nw, w13, w2, attn_nw)

    if tokens_p != tokens or dim_p != dim:
        out = out[:tokens, :dim]
    return out.reshape(B, S, dim)
